# baseline (speedup 1.0000x reference)
import functools

import jax
import jax.numpy as jnp
from jax import lax
from jax.experimental import pallas as pl
from jax.experimental.pallas import tpu as pltpu
from jax.experimental.pallas import tpu_sc as plsc


def kernel(x, pos_weight):
    B, L, H = x.shape
    NW = 32
    per_w = (L * H) // NW
    C = 24576
    n_chunks = per_w // C

    mesh = plsc.VectorSubcoreMesh(core_axis_name="c", subcore_axis_name="s")

    @functools.partial(
        pl.kernel,
        mesh=mesh,
        out_type=jax.ShapeDtypeStruct((B * L * H,), jnp.float32),
        scratch_types=[
            pltpu.VMEM((C,), jnp.float32),
            pltpu.VMEM((C,), jnp.float32),
        ],
    )
    def k(x_hbm, w_hbm, o_hbm, wv, xv):
        cid = lax.axis_index("c")
        sid = lax.axis_index("s")
        wid = sid * 2 + cid
        base = wid * per_w

        def chunk_body(t, carry):
            off = base + t * C
            pltpu.sync_copy(w_hbm.at[pl.ds(off, C)], wv)

            def batch_body(b, carry2):
                xoff = b * (L * H) + off
                pltpu.sync_copy(x_hbm.at[pl.ds(xoff, C)], xv)

                def add_body(i, carry3):
                    s = pl.ds(i * 16, 16)
                    xv[s] = xv[s] + wv[s]
                    return carry3

                lax.fori_loop(0, C // 16, add_body, 0)
                pltpu.sync_copy(xv, o_hbm.at[pl.ds(xoff, C)])
                return carry2

            lax.fori_loop(0, B, batch_body, 0)
            return carry

        lax.fori_loop(0, n_chunks, chunk_body, 0)

    out = k(x.reshape(-1), pos_weight.reshape(-1))
    return out.reshape(B, L, H)


# speedup vs baseline: 1.4233x; 1.4233x over previous
import functools

import jax
import jax.numpy as jnp
from jax import lax
from jax.experimental import pallas as pl
from jax.experimental.pallas import tpu as pltpu
from jax.experimental.pallas import tpu_sc as plsc


def kernel(x, pos_weight):
    B, L, H = x.shape
    NW = 32
    per_w = (L * H) // NW
    C = 24576
    n_chunks = per_w // C

    mesh = plsc.VectorSubcoreMesh(core_axis_name="c", subcore_axis_name="s")

    @functools.partial(
        pl.kernel,
        mesh=mesh,
        out_type=jax.ShapeDtypeStruct((B * L * H,), jnp.float32),
        scratch_types=[
            pltpu.VMEM((C,), jnp.float32),
            pltpu.VMEM((C,), jnp.float32),
        ],
    )
    def k(x_hbm, w_hbm, o_hbm, wv, xv):
        cid = lax.axis_index("c")
        sid = lax.axis_index("s")
        wid = sid * 2 + cid
        base = wid * per_w

        def chunk_body(t, carry):
            off = base + t * C
            pltpu.sync_copy(w_hbm.at[pl.ds(off, C)], wv)

            def batch_body(b, carry2):
                xoff = b * (L * H) + off
                pltpu.sync_copy(x_hbm.at[pl.ds(xoff, C)], xv)

                def add_body(i, carry3):
                    base_i = i * 256
                    for j in range(16):
                        s = pl.ds(base_i + j * 16, 16)
                        xv[s] = xv[s] + wv[s]
                    return carry3

                lax.fori_loop(0, C // 256, add_body, 0)
                pltpu.sync_copy(xv, o_hbm.at[pl.ds(xoff, C)])
                return carry2

            lax.fori_loop(0, B, batch_body, 0)
            return carry

        lax.fori_loop(0, n_chunks, chunk_body, 0)

    out = k(x.reshape(-1), pos_weight.reshape(-1))
    return out.reshape(B, L, H)


# speedup vs baseline: 3.1647x; 2.2236x over previous
import functools

import jax
import jax.numpy as jnp
from jax import lax
from jax.experimental import pallas as pl
from jax.experimental.pallas import tpu as pltpu
from jax.experimental.pallas import tpu_sc as plsc


def kernel(x, pos_weight):
    B, L, H = x.shape
    NW = 32
    rows_per_w = L // NW
    R = 32
    n_chunks = rows_per_w // R
    n_col = H // 16

    mesh = plsc.VectorSubcoreMesh(core_axis_name="c", subcore_axis_name="s")

    @functools.partial(
        pl.kernel,
        mesh=mesh,
        out_type=jax.ShapeDtypeStruct((B, L, H), jnp.float32),
        scratch_types=[
            pltpu.VMEM((R, H), jnp.float32),
            pltpu.VMEM((R, H), jnp.float32),
        ],
    )
    def k(x_hbm, w_hbm, o_hbm, wv, xv):
        cid = lax.axis_index("c")
        sid = lax.axis_index("s")
        wid = sid * 2 + cid
        base = wid * rows_per_w

        def chunk_body(t, carry):
            l0 = base + t * R
            pltpu.sync_copy(w_hbm.at[pl.ds(l0, R)], wv)

            def batch_body(b, carry2):
                pltpu.sync_copy(x_hbm.at[b, pl.ds(l0, R)], xv)

                def add_body(r, carry3):
                    for c in range(n_col):
                        s = pl.ds(c * 16, 16)
                        xv[r, s] = xv[r, s] + wv[r, s]
                    return carry3

                lax.fori_loop(0, R, add_body, 0)
                pltpu.sync_copy(xv, o_hbm.at[b, pl.ds(l0, R)])
                return carry2

            lax.fori_loop(0, B, batch_body, 0)
            return carry

        lax.fori_loop(0, n_chunks, chunk_body, 0)

    return k(x, pos_weight)


# speedup vs baseline: 4.9022x; 1.5490x over previous
import functools

import jax
import jax.numpy as jnp
from jax import lax
from jax.experimental import pallas as pl
from jax.experimental.pallas import tpu as pltpu
from jax.experimental.pallas import tpu_sc as plsc


def kernel(x, pos_weight):
    B, L, H = x.shape
    NW = 32
    rows_per_w = L // NW
    R = 32
    n_chunks = rows_per_w // R
    n_col = H // 16
    n_steps = n_chunks * B

    mesh = plsc.VectorSubcoreMesh(core_axis_name="c", subcore_axis_name="s")

    @functools.partial(
        pl.kernel,
        mesh=mesh,
        out_type=jax.ShapeDtypeStruct((B, L, H), jnp.float32),
        scratch_types=[
            pltpu.VMEM((R, H), jnp.float32),
            pltpu.VMEM((R, H), jnp.float32),
            pltpu.VMEM((R, H), jnp.float32),
            pltpu.VMEM((R, H), jnp.float32),
            pltpu.VMEM((R, H), jnp.float32),
            pltpu.SemaphoreType.DMA,
            pltpu.SemaphoreType.DMA,
            pltpu.SemaphoreType.DMA,
            pltpu.SemaphoreType.DMA,
            pltpu.SemaphoreType.DMA,
            pltpu.SemaphoreType.DMA,
            pltpu.SemaphoreType.DMA,
            pltpu.SemaphoreType.DMA,
        ],
    )
    def k(x_hbm, w_hbm, o_hbm, xv0, xv1, xv2, wv0, wv1,
          si0, si1, si2, so0, so1, so2, sw0, sw1):
        cid = lax.axis_index("c")
        sid = lax.axis_index("s")
        wid = sid * 2 + cid
        base = wid * rows_per_w

        xvs, wvs = [xv0, xv1, xv2], [wv0, wv1]
        sins, souts, sws = [si0, si1, si2], [so0, so1, so2], [sw0, sw1]

        def w_copy(t):
            l0 = base + t * R
            return pltpu.make_async_copy(
                w_hbm.at[pl.ds(l0, R)], wvs[t % 2], sws[t % 2])

        def in_copy(s):
            t, b, p = s // B, s % B, s % 3
            l0 = base + t * R
            return pltpu.make_async_copy(
                x_hbm.at[b, pl.ds(l0, R)], xvs[p], sins[p])

        def out_copy(s):
            t, b, p = s // B, s % B, s % 3
            l0 = base + t * R
            return pltpu.make_async_copy(
                xvs[p], o_hbm.at[b, pl.ds(l0, R)], souts[p])

        w_copy(0).start()
        in_copy(0).start()

        for s in range(n_steps):
            t, b, p = s // B, s % B, s % 3
            in_copy(s).wait()
            if b == 0:
                w_copy(t).wait()
                if t + 1 < n_chunks:
                    w_copy(t + 1).start()
            if s + 1 < n_steps:
                if s - 2 >= 0:
                    out_copy(s - 2).wait()
                in_copy(s + 1).start()

            xv, wv = xvs[p], wvs[t % 2]

            def add_body(r, carry, xv=xv, wv=wv):
                for c in range(n_col):
                    sl = pl.ds(c * 16, 16)
                    xv[r, sl] = xv[r, sl] + wv[r, sl]
                return carry

            lax.fori_loop(0, R, add_body, 0)
            out_copy(s).start()

        out_copy(n_steps - 2).wait()
        out_copy(n_steps - 1).wait()

    return k(x, pos_weight)


# speedup vs baseline: 4.9535x; 1.0105x over previous
import functools

import jax
import jax.numpy as jnp
from jax import lax
from jax.experimental import pallas as pl
from jax.experimental.pallas import tpu as pltpu
from jax.experimental.pallas import tpu_sc as plsc


def kernel(x, pos_weight):
    B, L, H = x.shape
    NW = 32
    rows_per_w = L // NW
    R = 32
    n_chunks = rows_per_w // R
    n_col = H // 16
    n_steps = n_chunks * B

    mesh = plsc.VectorSubcoreMesh(core_axis_name="c", subcore_axis_name="s")

    @functools.partial(
        pl.kernel,
        mesh=mesh,
        out_type=jax.ShapeDtypeStruct((B, L, H), jnp.float32),
        scratch_types=[
            pltpu.VMEM((R, H), jnp.float32),
            pltpu.VMEM((R, H), jnp.float32),
            pltpu.VMEM((R, H), jnp.float32),
            pltpu.VMEM((R, H), jnp.float32),
            pltpu.VMEM((R, H), jnp.float32),
            pltpu.SemaphoreType.DMA,
            pltpu.SemaphoreType.DMA,
            pltpu.SemaphoreType.DMA,
            pltpu.SemaphoreType.DMA,
            pltpu.SemaphoreType.DMA,
            pltpu.SemaphoreType.DMA,
            pltpu.SemaphoreType.DMA,
            pltpu.SemaphoreType.DMA,
            pltpu.SemaphoreType.DMA,
        ],
    )
    def k(x_hbm, w_hbm, o_hbm, xv0, xv1, xv2, xv3, wv,
          si0, si1, si2, si3, so0, so1, so2, so3, sw):
        cid = lax.axis_index("c")
        sid = lax.axis_index("s")
        wid = sid * 2 + cid
        base = wid * rows_per_w

        xvs = [xv0, xv1, xv2, xv3]
        sins, souts = [si0, si1, si2, si3], [so0, so1, so2, so3]

        def w_copy(t):
            l0 = base + t * R
            return pltpu.make_async_copy(w_hbm.at[pl.ds(l0, R)], wv, sw)

        def in_copy(s):
            t, b = s // B, s % B
            l0 = base + t * R
            return pltpu.make_async_copy(
                x_hbm.at[b, pl.ds(l0, R)], xvs[b], sins[b])

        def out_copy(s):
            t, b = s // B, s % B
            l0 = base + t * R
            return pltpu.make_async_copy(
                xvs[b], o_hbm.at[b, pl.ds(l0, R)], souts[b])

        w_copy(0).start()
        in_copy(0).start()
        in_copy(1).start()

        for s in range(n_steps):
            t, b = s // B, s % B
            in_copy(s).wait()
            if b == 0:
                w_copy(t).wait()
            if s + 2 < n_steps:
                if s - 2 >= 0:
                    out_copy(s - 2).wait()
                in_copy(s + 2).start()

            xv = xvs[b]

            def add_body(r, carry, xv=xv):
                for c in range(n_col):
                    sl = pl.ds(c * 16, 16)
                    xv[r, sl] = xv[r, sl] + wv[r, sl]
                return carry

            lax.fori_loop(0, R, add_body, 0)
            out_copy(s).start()
            if b == B - 1 and t + 1 < n_chunks:
                w_copy(t + 1).start()

        out_copy(n_steps - 2).wait()
        out_copy(n_steps - 1).wait()

    return k(x, pos_weight)


# speedup vs baseline: 5.0530x; 1.0201x over previous
import functools

import jax
import jax.numpy as jnp
from jax import lax
from jax.experimental import pallas as pl
from jax.experimental.pallas import tpu as pltpu
from jax.experimental.pallas import tpu_sc as plsc


def kernel(x, pos_weight):
    B, L, H = x.shape
    NW = 32
    rows_per_w = L // NW
    R = 32
    n_chunks = rows_per_w // R
    n_col = H // 16

    mesh = plsc.VectorSubcoreMesh(core_axis_name="c", subcore_axis_name="s")

    @functools.partial(
        pl.kernel,
        mesh=mesh,
        out_type=jax.ShapeDtypeStruct((B, L, H), jnp.float32),
        scratch_types=[
            pltpu.VMEM((R, H), jnp.float32),
            pltpu.VMEM((R, H), jnp.float32),
            pltpu.VMEM((R, H), jnp.float32),
            pltpu.VMEM((R, H), jnp.float32),
            pltpu.VMEM((R, H), jnp.float32),
            pltpu.SemaphoreType.DMA,
            pltpu.SemaphoreType.DMA,
            pltpu.SemaphoreType.DMA,
            pltpu.SemaphoreType.DMA,
            pltpu.SemaphoreType.DMA,
            pltpu.SemaphoreType.DMA,
            pltpu.SemaphoreType.DMA,
            pltpu.SemaphoreType.DMA,
            pltpu.SemaphoreType.DMA,
        ],
    )
    def k(x_hbm, w_hbm, o_hbm, xv0, xv1, xv2, xv3, wv,
          si0, si1, si2, si3, so0, so1, so2, so3, sw):
        cid = lax.axis_index("c")
        sid = lax.axis_index("s")
        wid = sid * 2 + cid
        base = wid * rows_per_w

        xvs = [xv0, xv1, xv2, xv3]
        sins, souts = [si0, si1, si2, si3], [so0, so1, so2, so3]

        def w_copy(t):
            return pltpu.make_async_copy(
                w_hbm.at[pl.ds(base + t * R, R)], wv, sw)

        def in_copy(t, b):
            return pltpu.make_async_copy(
                x_hbm.at[b, pl.ds(base + t * R, R)], xvs[b], sins[b])

        def out_copy(t, b):
            return pltpu.make_async_copy(
                xvs[b], o_hbm.at[b, pl.ds(base + t * R, R)], souts[b])

        def step(t, b, first=False, last=False):
            in_copy(t, b).wait()
            if b == 0:
                w_copy(t).wait()
            if b < 2:
                if not first:
                    out_copy(t - 1, b + 2).wait()
                in_copy(t, b + 2).start()
            else:
                out_copy(t, b - 2).wait()
                if not last:
                    in_copy(t + 1, b - 2).start()

            xv = xvs[b]

            @plsc.parallel_loop(0, R, 1, unroll=2)
            def add_body(r, xv=xv):
                for c in range(n_col):
                    sl = pl.ds(c * 16, 16)
                    xv[r, sl] = xv[r, sl] + wv[r, sl]

            out_copy(t, b).start()
            if b == B - 1 and not last:
                w_copy(t + 1).start()

        w_copy(0).start()
        in_copy(0, 0).start()
        in_copy(0, 1).start()

        for b in range(B):
            step(0, b, first=True)

        def mid_body(t, carry):
            for b in range(B):
                step(t, b)
            return carry

        lax.fori_loop(1, n_chunks - 1, mid_body, 0)

        for b in range(B):
            step(n_chunks - 1, b, last=True)

        out_copy(n_chunks - 1, 2).wait()
        out_copy(n_chunks - 1, 3).wait()

    return k(x, pos_weight)
